# baseline (speedup 1.0000x reference)
import functools

import jax
import jax.numpy as jnp
from jax import lax
from jax.experimental import pallas as pl
from jax.experimental.pallas import tpu as pltpu, tpu_sc as plsc

N = 10000
NP = 10240
E = 320000
IN_DIM = 128
HID_DIM = 128
OUT_DIM = 64

NC = 2
NS = 16
EROW = 128
ROWS = 2560
EPAD = ROWS * EROW
STRIPE = NP // NS

BN = 1024
GRID = NP // BN

@functools.lru_cache(maxsize=None)
def _mesh():
    return plsc.VectorSubcoreMesh(
        core_axis_name="c", subcore_axis_name="s", num_cores=NC, num_subcores=NS
    )


@functools.lru_cache(maxsize=None)
def _make_deg_kernel():
    RD = ROWS // (NC * NS)

    @functools.partial(
        pl.kernel,
        mesh=_mesh(),
        compiler_params=pltpu.CompilerParams(use_tc_tiling_on_sc=False),
        out_type=jax.ShapeDtypeStruct((NC, NP, 16), jnp.float32),
        scratch_types=[
            pltpu.VMEM((RD, EROW), jnp.int32),
            pltpu.VMEM((EROW, 16), jnp.float32),
            pltpu.VMEM_SHARED((NP, 16), jnp.float32),
            [pltpu.SemaphoreType.DMA] * 4,
        ],
    )
    def deg_kernel(dst_hbm, ones_hbm, zeros_hbm, out, dst_v, ones_v, acc, ssem):
        cid = lax.axis_index("c")
        sid = lax.axis_index("s")
        r0 = sid * STRIPE
        pltpu.sync_copy(zeros_hbm, acc.at[pl.ds(r0, STRIPE)])
        e0 = (cid * NS + sid) * RD
        pltpu.sync_copy(dst_hbm.at[pl.ds(e0, RD)], dst_v)
        pltpu.sync_copy(ones_hbm, ones_v)
        plsc.subcore_barrier()

        def scat(j, b):
            return pltpu.make_async_copy(ones_v, acc.at[dst_v.at[j]], ssem[b])

        @pl.loop(0, RD, step=4)
        def _(j0):
            for b in range(4):
                j = j0 + b

                @pl.when(j >= 4)
                def _():
                    scat(0, b).wait()

                scat(j, b).start(add=True)

        for b in range(4):
            scat(0, b).wait()
        plsc.subcore_barrier()
        pltpu.sync_copy(acc.at[pl.ds(r0, STRIPE)], out.at[cid].at[pl.ds(r0, STRIPE)])

    return deg_kernel


@functools.lru_cache(maxsize=None)
def _make_edge_kernel(H):
    RT = ROWS // NS

    NB = 8
    AH = NB // 2
    IC = 16
    NCH = RT // IC

    @functools.partial(
        pl.kernel,
        mesh=_mesh(),
        compiler_params=pltpu.CompilerParams(use_tc_tiling_on_sc=False),
        out_type=jax.ShapeDtypeStruct((NC, NP, H), jnp.bfloat16),
        scratch_types=[
            pltpu.VMEM((2, IC, EROW), jnp.int32),
            pltpu.VMEM((2, IC, EROW), jnp.int32),
            pltpu.VMEM((NB, EROW, H), jnp.bfloat16),
            pltpu.VMEM_SHARED((NP, H), jnp.bfloat16),
            pltpu.VMEM_SHARED((NP, H), jnp.bfloat16),
            [pltpu.SemaphoreType.DMA] * NB,
            [pltpu.SemaphoreType.DMA] * NB,
            [pltpu.SemaphoreType.DMA] * 2,
        ],
    )
    def edge_kernel(tbl, src_hbm, dst_hbm, out, src_v, dst_v, rows_v, acc, tbl_sh, gsem, ssem, isem):
        cid = lax.axis_index("c")
        sid = lax.axis_index("s")
        tblc = tbl.at[cid]
        e0 = sid * RT

        def idx_fetch(c, p):
            return (
                pltpu.make_async_copy(
                    src_hbm.at[pl.ds(e0 + c * IC, IC)], src_v.at[p], isem[p]
                ),
                pltpu.make_async_copy(
                    dst_hbm.at[pl.ds(e0 + c * IC, IC)], dst_v.at[p], isem[p]
                ),
            )

        def gather(p, j, b):
            return pltpu.make_async_copy(tbl_sh.at[src_v.at[p].at[j]], rows_v.at[b], gsem[b])

        def scatter(p, j, b):
            return pltpu.make_async_copy(rows_v.at[b], acc.at[dst_v.at[p].at[j]], ssem[b])

        r0 = sid * STRIPE
        pltpu.sync_copy(tblc.at[pl.ds(r0, STRIPE)], acc.at[pl.ds(r0, STRIPE)])
        pltpu.sync_copy(tblc.at[pl.ds(r0, STRIPE)], tbl_sh.at[pl.ds(r0, STRIPE)])
        for d in idx_fetch(0, 0):
            d.start()
        for d in idx_fetch(0, 0):
            d.wait()
        plsc.subcore_barrier()
        for b in range(AH):
            gather(0, b, b).start()

        for c in range(NCH):
            p = c % 2
            if c + 1 < NCH:
                for d in idx_fetch(c + 1, 1 - p):
                    d.start()

            @pl.loop(0, IC, step=NB)
            def _(j0):
                for b in range(NB):
                    j = j0 + b
                    gather(p, j, b).wait()
                    scatter(p, j, b).start(add=True)
                    jf = j + AH
                    bf = (b + AH) % NB

                    @pl.when(jf < IC)
                    def _():
                        @pl.when(jf >= NB)
                        def _():
                            scatter(p, 0, bf).wait()

                        gather(p, jf, bf).start()

            for b in range(NB):
                scatter(p, 0, b).wait()
            if c + 1 < NCH:
                for d in idx_fetch(c + 1, 1 - p):
                    d.wait()
                for b in range(AH):
                    gather(1 - p, b, b).start()

        plsc.subcore_barrier()
        pltpu.sync_copy(acc.at[pl.ds(r0, STRIPE)], out.at[cid].at[pl.ds(r0, STRIPE)])

    return edge_kernel


def _dinv_col(deg_ref, nrows):
    d = deg_ref[0, :nrows, 0:1] + deg_ref[1, :nrows, 0:1] + 1.0
    return lax.rsqrt(d)


def _tc_scale_matmul(x_ref, w_ref, deg_ref, out_ref):
    dinv = _dinv_col(deg_ref, N)
    h = jnp.dot(x_ref[...], w_ref[...], preferred_element_type=jnp.float32) * dinv
    h = h.astype(jnp.bfloat16)
    zpad = jnp.zeros((NP - N, HID_DIM // 2), jnp.bfloat16)
    out_ref[0] = jnp.concatenate([h[:, : HID_DIM // 2], zpad], axis=0)
    out_ref[1] = jnp.concatenate([h[:, HID_DIM // 2 :], zpad], axis=0)


def _tc_mid(t_ref, deg_ref, w_ref, b_ref, out_ref):
    dinv = _dinv_col(deg_ref, N)
    tmp = jnp.concatenate(
        [t_ref[0, :N].astype(jnp.float32), t_ref[1, :N].astype(jnp.float32)], axis=1
    )
    z = jnp.maximum(tmp * dinv + b_ref[...], 0.0)
    h = jnp.dot(z, w_ref[...], preferred_element_type=jnp.float32) * dinv
    h = h.astype(jnp.bfloat16)
    zpad = jnp.zeros((NP - N, OUT_DIM // 2), jnp.bfloat16)
    out_ref[0] = jnp.concatenate([h[:, : OUT_DIM // 2], zpad], axis=0)
    out_ref[1] = jnp.concatenate([h[:, OUT_DIM // 2 :], zpad], axis=0)


def _tc_final(t_ref, deg_ref, b_ref, out_ref):
    dinv = _dinv_col(deg_ref, N)
    o = jnp.concatenate(
        [t_ref[0, :N].astype(jnp.float32), t_ref[1, :N].astype(jnp.float32)], axis=1
    ) * dinv + b_ref[...]
    m = jnp.max(o, axis=1, keepdims=True)
    z = o - m
    out_ref[...] = z - jnp.log(jnp.sum(jnp.exp(z), axis=1, keepdims=True))


_scale_matmul_call = pl.pallas_call(
    _tc_scale_matmul,
    out_shape=jax.ShapeDtypeStruct((NC, NP, HID_DIM // 2), jnp.bfloat16),
)

_mid_call = pl.pallas_call(
    _tc_mid,
    out_shape=jax.ShapeDtypeStruct((NC, NP, OUT_DIM // 2), jnp.bfloat16),
)

_final_call = pl.pallas_call(
    _tc_final,
    out_shape=jax.ShapeDtypeStruct((N, OUT_DIM), jnp.float32),
)

def kernel(x, edge_index, W1, b1, W2, b2):
    ei = edge_index.astype(jnp.int32)
    pad = jnp.full((EPAD - E,), N, jnp.int32)
    src = jnp.concatenate([ei[0], pad]).reshape(ROWS, EROW)
    dst = jnp.concatenate([ei[1], pad]).reshape(ROWS, EROW)
    ones16 = jnp.ones((EROW, 16), jnp.float32)
    zeros16 = jnp.zeros((STRIPE, 16), jnp.float32)

    degp = _make_deg_kernel()(dst, ones16, zeros16)
    tbl1 = _scale_matmul_call(x, W1, degp)
    acc1 = _make_edge_kernel(HID_DIM // 2)(tbl1, src, dst)
    tbl2 = _mid_call(acc1, degp, W2, b1.reshape(1, HID_DIM))
    acc2 = _make_edge_kernel(OUT_DIM // 2)(tbl2, src, dst)
    return _final_call(acc2, degp, b2.reshape(1, OUT_DIM))


# speedup vs baseline: 1.0404x; 1.0404x over previous
import functools

import jax
import jax.numpy as jnp
from jax import lax
from jax.experimental import pallas as pl
from jax.experimental.pallas import tpu as pltpu, tpu_sc as plsc

N = 10000
NP = 10240
E = 320000
IN_DIM = 128
HID_DIM = 128
OUT_DIM = 64

NC = 2
NS = 16
EROW = 128
ROWS = 2560
EPAD = ROWS * EROW
STRIPE = NP // NS

BN = 1024
GRID = NP // BN

@functools.lru_cache(maxsize=None)
def _mesh():
    return plsc.VectorSubcoreMesh(
        core_axis_name="c", subcore_axis_name="s", num_cores=NC, num_subcores=NS
    )


@functools.lru_cache(maxsize=None)
def _make_deg_kernel():
    RD = ROWS // (NC * NS)

    @functools.partial(
        pl.kernel,
        mesh=_mesh(),
        compiler_params=pltpu.CompilerParams(use_tc_tiling_on_sc=False),
        out_type=jax.ShapeDtypeStruct((NC, NP, 16), jnp.float32),
        scratch_types=[
            pltpu.VMEM((RD, EROW), jnp.int32),
            pltpu.VMEM((EROW, 16), jnp.float32),
            pltpu.VMEM_SHARED((NP, 16), jnp.float32),
            [pltpu.SemaphoreType.DMA] * 4,
        ],
    )
    def deg_kernel(dst_hbm, ones_hbm, zeros_hbm, out, dst_v, ones_v, acc, ssem):
        cid = lax.axis_index("c")
        sid = lax.axis_index("s")
        r0 = sid * STRIPE
        pltpu.sync_copy(zeros_hbm, acc.at[pl.ds(r0, STRIPE)])
        e0 = (cid * NS + sid) * RD
        pltpu.sync_copy(dst_hbm.at[pl.ds(e0, RD)], dst_v)
        pltpu.sync_copy(ones_hbm, ones_v)
        plsc.subcore_barrier()

        def scat(j, b):
            return pltpu.make_async_copy(ones_v, acc.at[dst_v.at[j]], ssem[b])

        @pl.loop(0, RD, step=4)
        def _(j0):
            for b in range(4):
                j = j0 + b

                @pl.when(j >= 4)
                def _():
                    scat(0, b).wait()

                scat(j, b).start(add=True)

        for b in range(4):
            scat(0, b).wait()
        plsc.subcore_barrier()
        pltpu.sync_copy(acc.at[pl.ds(r0, STRIPE)], out.at[cid].at[pl.ds(r0, STRIPE)])

    return deg_kernel


@functools.lru_cache(maxsize=None)
def _make_edge_kernel(H):
    RT = ROWS // NS

    NB = 4
    AH = NB // 2
    IC = 32
    NCH = RT // IC

    @functools.partial(
        pl.kernel,
        mesh=_mesh(),
        compiler_params=pltpu.CompilerParams(use_tc_tiling_on_sc=False),
        out_type=jax.ShapeDtypeStruct((NC, NP, H), jnp.bfloat16),
        scratch_types=[
            pltpu.VMEM((2, IC, EROW), jnp.int32),
            pltpu.VMEM((2, IC, EROW), jnp.int32),
            pltpu.VMEM((NB, EROW, H), jnp.bfloat16),
            pltpu.VMEM_SHARED((NP, H), jnp.bfloat16),
            pltpu.VMEM_SHARED((NP, H), jnp.bfloat16),
            [pltpu.SemaphoreType.DMA] * NB,
            [pltpu.SemaphoreType.DMA] * NB,
            [pltpu.SemaphoreType.DMA] * 2,
        ],
    )
    def edge_kernel(tbl, src_hbm, dst_hbm, out, src_v, dst_v, rows_v, acc, tbl_sh, gsem, ssem, isem):
        cid = lax.axis_index("c")
        sid = lax.axis_index("s")
        tblc = tbl.at[cid]
        e0 = sid * RT

        def idx_fetch(c, p):
            return (
                pltpu.make_async_copy(
                    src_hbm.at[pl.ds(e0 + c * IC, IC)], src_v.at[p], isem[p]
                ),
                pltpu.make_async_copy(
                    dst_hbm.at[pl.ds(e0 + c * IC, IC)], dst_v.at[p], isem[p]
                ),
            )

        def gather(p, j, b):
            return pltpu.make_async_copy(tbl_sh.at[src_v.at[p].at[j]], rows_v.at[b], gsem[b])

        def scatter(p, j, b):
            return pltpu.make_async_copy(rows_v.at[b], acc.at[dst_v.at[p].at[j]], ssem[b])

        r0 = sid * STRIPE
        pltpu.sync_copy(tblc.at[pl.ds(r0, STRIPE)], acc.at[pl.ds(r0, STRIPE)])
        pltpu.sync_copy(tblc.at[pl.ds(r0, STRIPE)], tbl_sh.at[pl.ds(r0, STRIPE)])
        for d in idx_fetch(0, 0):
            d.start()
        for d in idx_fetch(0, 0):
            d.wait()
        plsc.subcore_barrier()
        for b in range(AH):
            gather(0, b, b).start()

        for c in range(NCH):
            p = c % 2
            if c + 1 < NCH:
                for d in idx_fetch(c + 1, 1 - p):
                    d.start()

            @pl.loop(0, IC, step=NB)
            def _(j0):
                for b in range(NB):
                    j = j0 + b
                    gather(p, j, b).wait()
                    scatter(p, j, b).start(add=True)
                    jf = j + AH
                    bf = (b + AH) % NB

                    @pl.when(jf < IC)
                    def _():
                        @pl.when(jf >= NB)
                        def _():
                            scatter(p, 0, bf).wait()

                        gather(p, jf, bf).start()

            for b in range(NB):
                scatter(p, 0, b).wait()
            if c + 1 < NCH:
                for d in idx_fetch(c + 1, 1 - p):
                    d.wait()
                for b in range(AH):
                    gather(1 - p, b, b).start()

        plsc.subcore_barrier()
        pltpu.sync_copy(acc.at[pl.ds(r0, STRIPE)], out.at[cid].at[pl.ds(r0, STRIPE)])

    return edge_kernel


def _dinv_col(deg_ref, nrows):
    d = deg_ref[0, :nrows, 0:1] + deg_ref[1, :nrows, 0:1] + 1.0
    return lax.rsqrt(d)


def _tc_scale_matmul(x_ref, w_ref, deg_ref, out_ref):
    dinv = _dinv_col(deg_ref, N)
    h = jnp.dot(x_ref[...], w_ref[...], preferred_element_type=jnp.float32) * dinv
    h = h.astype(jnp.bfloat16)
    zpad = jnp.zeros((NP - N, HID_DIM // 2), jnp.bfloat16)
    out_ref[0] = jnp.concatenate([h[:, : HID_DIM // 2], zpad], axis=0)
    out_ref[1] = jnp.concatenate([h[:, HID_DIM // 2 :], zpad], axis=0)


def _tc_mid(t_ref, deg_ref, w_ref, b_ref, out_ref):
    dinv = _dinv_col(deg_ref, N)
    tmp = jnp.concatenate(
        [t_ref[0, :N].astype(jnp.float32), t_ref[1, :N].astype(jnp.float32)], axis=1
    )
    z = jnp.maximum(tmp * dinv + b_ref[...], 0.0)
    h = jnp.dot(z, w_ref[...], preferred_element_type=jnp.float32) * dinv
    h = h.astype(jnp.bfloat16)
    zpad = jnp.zeros((NP - N, OUT_DIM // 2), jnp.bfloat16)
    out_ref[0] = jnp.concatenate([h[:, : OUT_DIM // 2], zpad], axis=0)
    out_ref[1] = jnp.concatenate([h[:, OUT_DIM // 2 :], zpad], axis=0)


def _tc_final(t_ref, deg_ref, b_ref, out_ref):
    dinv = _dinv_col(deg_ref, N)
    o = jnp.concatenate(
        [t_ref[0, :N].astype(jnp.float32), t_ref[1, :N].astype(jnp.float32)], axis=1
    ) * dinv + b_ref[...]
    m = jnp.max(o, axis=1, keepdims=True)
    z = o - m
    out_ref[...] = z - jnp.log(jnp.sum(jnp.exp(z), axis=1, keepdims=True))


_scale_matmul_call = pl.pallas_call(
    _tc_scale_matmul,
    out_shape=jax.ShapeDtypeStruct((NC, NP, HID_DIM // 2), jnp.bfloat16),
)

_mid_call = pl.pallas_call(
    _tc_mid,
    out_shape=jax.ShapeDtypeStruct((NC, NP, OUT_DIM // 2), jnp.bfloat16),
)

_final_call = pl.pallas_call(
    _tc_final,
    out_shape=jax.ShapeDtypeStruct((N, OUT_DIM), jnp.float32),
)

def kernel(x, edge_index, W1, b1, W2, b2):
    ei = edge_index.astype(jnp.int32)
    pad = jnp.full((EPAD - E,), N, jnp.int32)
    src = jnp.concatenate([ei[0], pad]).reshape(ROWS, EROW)
    dst = jnp.concatenate([ei[1], pad]).reshape(ROWS, EROW)
    ones16 = jnp.ones((EROW, 16), jnp.float32)
    zeros16 = jnp.zeros((STRIPE, 16), jnp.float32)

    degp = _make_deg_kernel()(dst, ones16, zeros16)
    tbl1 = _scale_matmul_call(x, W1, degp)
    acc1 = _make_edge_kernel(HID_DIM // 2)(tbl1, src, dst)
    tbl2 = _mid_call(acc1, degp, W2, b1.reshape(1, HID_DIM))
    acc2 = _make_edge_kernel(OUT_DIM // 2)(tbl2, src, dst)
    return _final_call(acc2, degp, b2.reshape(1, OUT_DIM))


# speedup vs baseline: 1.0447x; 1.0041x over previous
import functools

import jax
import jax.numpy as jnp
from jax import lax
from jax.experimental import pallas as pl
from jax.experimental.pallas import tpu as pltpu, tpu_sc as plsc

N = 10000
NP = 10240
E = 320000
IN_DIM = 128
HID_DIM = 128
OUT_DIM = 64

NC = 2
NS = 16
EROW = 128
ROWS = 2560
EPAD = ROWS * EROW
STRIPE = NP // NS

BN = 1024
GRID = NP // BN

@functools.lru_cache(maxsize=None)
def _mesh():
    return plsc.VectorSubcoreMesh(
        core_axis_name="c", subcore_axis_name="s", num_cores=NC, num_subcores=NS
    )


@functools.lru_cache(maxsize=None)
def _make_deg_kernel():
    RD = ROWS // (NC * NS)

    @functools.partial(
        pl.kernel,
        mesh=_mesh(),
        compiler_params=pltpu.CompilerParams(use_tc_tiling_on_sc=False),
        out_type=jax.ShapeDtypeStruct((NC, NP, 16), jnp.float32),
        scratch_types=[
            pltpu.VMEM((RD, EROW), jnp.int32),
            pltpu.VMEM((EROW, 16), jnp.float32),
            pltpu.VMEM_SHARED((NP, 16), jnp.float32),
            [pltpu.SemaphoreType.DMA] * 4,
        ],
    )
    def deg_kernel(dst_hbm, ones_hbm, zeros_hbm, out, dst_v, ones_v, acc, ssem):
        cid = lax.axis_index("c")
        sid = lax.axis_index("s")
        r0 = sid * STRIPE
        pltpu.sync_copy(zeros_hbm, acc.at[pl.ds(r0, STRIPE)])
        e0 = (cid * NS + sid) * RD
        pltpu.sync_copy(dst_hbm.at[pl.ds(e0, RD)], dst_v)
        pltpu.sync_copy(ones_hbm, ones_v)
        plsc.subcore_barrier()

        def scat(j, b):
            return pltpu.make_async_copy(ones_v, acc.at[dst_v.at[j]], ssem[b])

        @pl.loop(0, RD, step=4)
        def _(j0):
            for b in range(4):
                j = j0 + b

                @pl.when(j >= 4)
                def _():
                    scat(0, b).wait()

                scat(j, b).start(add=True)

        for b in range(4):
            scat(0, b).wait()
        plsc.subcore_barrier()
        pltpu.sync_copy(acc.at[pl.ds(r0, STRIPE)], out.at[cid].at[pl.ds(r0, STRIPE)])

    return deg_kernel


@functools.lru_cache(maxsize=None)
def _make_edge_kernel(H):
    RT = ROWS // NS

    NB = 4
    AH = NB // 2
    IC = 40
    NCH = RT // IC

    @functools.partial(
        pl.kernel,
        mesh=_mesh(),
        compiler_params=pltpu.CompilerParams(use_tc_tiling_on_sc=False),
        out_type=jax.ShapeDtypeStruct((NC, NP, H), jnp.bfloat16),
        scratch_types=[
            pltpu.VMEM((2, IC, EROW), jnp.int32),
            pltpu.VMEM((2, IC, EROW), jnp.int32),
            pltpu.VMEM((NB, EROW, H), jnp.bfloat16),
            pltpu.VMEM_SHARED((NP, H), jnp.bfloat16),
            pltpu.VMEM_SHARED((NP, H), jnp.bfloat16),
            [pltpu.SemaphoreType.DMA] * NB,
            [pltpu.SemaphoreType.DMA] * NB,
            [pltpu.SemaphoreType.DMA] * 2,
        ],
    )
    def edge_kernel(tbl, src_hbm, dst_hbm, out, src_v, dst_v, rows_v, acc, tbl_sh, gsem, ssem, isem):
        cid = lax.axis_index("c")
        sid = lax.axis_index("s")
        tblc = tbl.at[cid]
        e0 = sid * RT

        def idx_fetch(c, p):
            return (
                pltpu.make_async_copy(
                    src_hbm.at[pl.ds(e0 + c * IC, IC)], src_v.at[p], isem[p]
                ),
                pltpu.make_async_copy(
                    dst_hbm.at[pl.ds(e0 + c * IC, IC)], dst_v.at[p], isem[p]
                ),
            )

        def gather(p, j, b):
            return pltpu.make_async_copy(tbl_sh.at[src_v.at[p].at[j]], rows_v.at[b], gsem[b])

        def scatter(p, j, b):
            return pltpu.make_async_copy(rows_v.at[b], acc.at[dst_v.at[p].at[j]], ssem[b])

        r0 = sid * STRIPE
        pltpu.sync_copy(tblc.at[pl.ds(r0, STRIPE)], acc.at[pl.ds(r0, STRIPE)])
        pltpu.sync_copy(tblc.at[pl.ds(r0, STRIPE)], tbl_sh.at[pl.ds(r0, STRIPE)])
        for d in idx_fetch(0, 0):
            d.start()
        for d in idx_fetch(0, 0):
            d.wait()
        plsc.subcore_barrier()
        for b in range(AH):
            gather(0, b, b).start()

        for c in range(NCH):
            p = c % 2
            if c + 1 < NCH:
                for d in idx_fetch(c + 1, 1 - p):
                    d.start()

            @pl.loop(0, IC, step=NB)
            def _(j0):
                for b in range(NB):
                    j = j0 + b
                    gather(p, j, b).wait()
                    scatter(p, j, b).start(add=True)
                    jf = j + AH
                    bf = (b + AH) % NB

                    @pl.when(jf < IC)
                    def _():
                        @pl.when(jf >= NB)
                        def _():
                            scatter(p, 0, bf).wait()

                        gather(p, jf, bf).start()

            for b in range(NB):
                scatter(p, 0, b).wait()
            if c + 1 < NCH:
                for d in idx_fetch(c + 1, 1 - p):
                    d.wait()
                for b in range(AH):
                    gather(1 - p, b, b).start()

        plsc.subcore_barrier()
        pltpu.sync_copy(acc.at[pl.ds(r0, STRIPE)], out.at[cid].at[pl.ds(r0, STRIPE)])

    return edge_kernel


def _dinv_col(deg_ref, nrows):
    d = deg_ref[0, :nrows, 0:1] + deg_ref[1, :nrows, 0:1] + 1.0
    return lax.rsqrt(d)


def _tc_scale_matmul(x_ref, w_ref, deg_ref, out_ref):
    dinv = _dinv_col(deg_ref, N)
    h = jnp.dot(x_ref[...], w_ref[...], preferred_element_type=jnp.float32) * dinv
    h = h.astype(jnp.bfloat16)
    zpad = jnp.zeros((NP - N, HID_DIM // 2), jnp.bfloat16)
    out_ref[0] = jnp.concatenate([h[:, : HID_DIM // 2], zpad], axis=0)
    out_ref[1] = jnp.concatenate([h[:, HID_DIM // 2 :], zpad], axis=0)


def _tc_mid(t_ref, deg_ref, w_ref, b_ref, out_ref):
    dinv = _dinv_col(deg_ref, N)
    tmp = jnp.concatenate(
        [t_ref[0, :N].astype(jnp.float32), t_ref[1, :N].astype(jnp.float32)], axis=1
    )
    z = jnp.maximum(tmp * dinv + b_ref[...], 0.0)
    h = jnp.dot(z, w_ref[...], preferred_element_type=jnp.float32) * dinv
    h = h.astype(jnp.bfloat16)
    zpad = jnp.zeros((NP - N, OUT_DIM // 2), jnp.bfloat16)
    out_ref[0] = jnp.concatenate([h[:, : OUT_DIM // 2], zpad], axis=0)
    out_ref[1] = jnp.concatenate([h[:, OUT_DIM // 2 :], zpad], axis=0)


def _tc_final(t_ref, deg_ref, b_ref, out_ref):
    dinv = _dinv_col(deg_ref, N)
    o = jnp.concatenate(
        [t_ref[0, :N].astype(jnp.float32), t_ref[1, :N].astype(jnp.float32)], axis=1
    ) * dinv + b_ref[...]
    m = jnp.max(o, axis=1, keepdims=True)
    z = o - m
    out_ref[...] = z - jnp.log(jnp.sum(jnp.exp(z), axis=1, keepdims=True))


_scale_matmul_call = pl.pallas_call(
    _tc_scale_matmul,
    out_shape=jax.ShapeDtypeStruct((NC, NP, HID_DIM // 2), jnp.bfloat16),
)

_mid_call = pl.pallas_call(
    _tc_mid,
    out_shape=jax.ShapeDtypeStruct((NC, NP, OUT_DIM // 2), jnp.bfloat16),
)

_final_call = pl.pallas_call(
    _tc_final,
    out_shape=jax.ShapeDtypeStruct((N, OUT_DIM), jnp.float32),
)

def kernel(x, edge_index, W1, b1, W2, b2):
    ei = edge_index.astype(jnp.int32)
    pad = jnp.full((EPAD - E,), N, jnp.int32)
    src = jnp.concatenate([ei[0], pad]).reshape(ROWS, EROW)
    dst = jnp.concatenate([ei[1], pad]).reshape(ROWS, EROW)
    ones16 = jnp.ones((EROW, 16), jnp.float32)
    zeros16 = jnp.zeros((STRIPE, 16), jnp.float32)

    degp = _make_deg_kernel()(dst, ones16, zeros16)
    tbl1 = _scale_matmul_call(x, W1, degp)
    acc1 = _make_edge_kernel(HID_DIM // 2)(tbl1, src, dst)
    tbl2 = _mid_call(acc1, degp, W2, b1.reshape(1, HID_DIM))
    acc2 = _make_edge_kernel(OUT_DIM // 2)(tbl2, src, dst)
    return _final_call(acc2, degp, b2.reshape(1, OUT_DIM))


# speedup vs baseline: 1.0557x; 1.0105x over previous
import functools

import jax
import jax.numpy as jnp
from jax import lax
from jax.experimental import pallas as pl
from jax.experimental.pallas import tpu as pltpu, tpu_sc as plsc

N = 10000
NP = 10240
E = 320000
IN_DIM = 128
HID_DIM = 128
OUT_DIM = 64

NC = 2
NS = 16
EROW = 128
ROWS = 2560
EPAD = ROWS * EROW
STRIPE = NP // NS

BN = 1024
GRID = NP // BN

@functools.lru_cache(maxsize=None)
def _mesh():
    return plsc.VectorSubcoreMesh(
        core_axis_name="c", subcore_axis_name="s", num_cores=NC, num_subcores=NS
    )


@functools.lru_cache(maxsize=None)
def _make_deg_kernel():
    RD = ROWS // (NC * NS)

    @functools.partial(
        pl.kernel,
        mesh=_mesh(),
        compiler_params=pltpu.CompilerParams(use_tc_tiling_on_sc=False),
        out_type=jax.ShapeDtypeStruct((NC, NP, 16), jnp.float32),
        scratch_types=[
            pltpu.VMEM((RD, EROW), jnp.int32),
            pltpu.VMEM((EROW, 16), jnp.float32),
            pltpu.VMEM_SHARED((NP, 16), jnp.float32),
            [pltpu.SemaphoreType.DMA] * 4,
        ],
    )
    def deg_kernel(dst_hbm, ones_hbm, zeros_hbm, out, dst_v, ones_v, acc, ssem):
        cid = lax.axis_index("c")
        sid = lax.axis_index("s")
        r0 = sid * STRIPE
        pltpu.sync_copy(zeros_hbm, acc.at[pl.ds(r0, STRIPE)])
        e0 = (cid * NS + sid) * RD
        pltpu.sync_copy(dst_hbm.at[pl.ds(e0, RD)], dst_v)
        pltpu.sync_copy(ones_hbm, ones_v)
        plsc.subcore_barrier()

        def scat(j, b):
            return pltpu.make_async_copy(ones_v, acc.at[dst_v.at[j]], ssem[b])

        @pl.loop(0, RD, step=4)
        def _(j0):
            for b in range(4):
                j = j0 + b

                @pl.when(j >= 4)
                def _():
                    scat(0, b).wait()

                scat(j, b).start(add=True)

        for b in range(4):
            scat(0, b).wait()
        plsc.subcore_barrier()
        pltpu.sync_copy(acc.at[pl.ds(r0, STRIPE)], out.at[cid].at[pl.ds(r0, STRIPE)])

    return deg_kernel


@functools.lru_cache(maxsize=None)
def _make_edge_kernel(H):
    RT = ROWS // NS

    NB = 4
    AH = NB // 2
    IC = 80
    NCH = RT // IC

    @functools.partial(
        pl.kernel,
        mesh=_mesh(),
        compiler_params=pltpu.CompilerParams(use_tc_tiling_on_sc=False),
        out_type=jax.ShapeDtypeStruct((NC, NP, H), jnp.bfloat16),
        scratch_types=[
            pltpu.VMEM((2, IC, EROW), jnp.int32),
            pltpu.VMEM((2, IC, EROW), jnp.int32),
            pltpu.VMEM((NB, EROW, H), jnp.bfloat16),
            pltpu.VMEM_SHARED((NP, H), jnp.bfloat16),
            pltpu.VMEM_SHARED((NP, H), jnp.bfloat16),
            [pltpu.SemaphoreType.DMA] * NB,
            [pltpu.SemaphoreType.DMA] * NB,
            [pltpu.SemaphoreType.DMA] * 2,
        ],
    )
    def edge_kernel(tbl, src_hbm, dst_hbm, out, src_v, dst_v, rows_v, acc, tbl_sh, gsem, ssem, isem):
        cid = lax.axis_index("c")
        sid = lax.axis_index("s")
        tblc = tbl.at[cid]
        e0 = sid * RT

        def idx_fetch(c, p):
            return (
                pltpu.make_async_copy(
                    src_hbm.at[pl.ds(e0 + c * IC, IC)], src_v.at[p], isem[p]
                ),
                pltpu.make_async_copy(
                    dst_hbm.at[pl.ds(e0 + c * IC, IC)], dst_v.at[p], isem[p]
                ),
            )

        def gather(p, j, b):
            return pltpu.make_async_copy(tbl_sh.at[src_v.at[p].at[j]], rows_v.at[b], gsem[b])

        def scatter(p, j, b):
            return pltpu.make_async_copy(rows_v.at[b], acc.at[dst_v.at[p].at[j]], ssem[b])

        r0 = sid * STRIPE
        pltpu.sync_copy(tblc.at[pl.ds(r0, STRIPE)], acc.at[pl.ds(r0, STRIPE)])
        pltpu.sync_copy(tblc.at[pl.ds(r0, STRIPE)], tbl_sh.at[pl.ds(r0, STRIPE)])
        for d in idx_fetch(0, 0):
            d.start()
        for d in idx_fetch(0, 0):
            d.wait()
        plsc.subcore_barrier()
        for b in range(AH):
            gather(0, b, b).start()

        for c in range(NCH):
            p = c % 2
            if c + 1 < NCH:
                for d in idx_fetch(c + 1, 1 - p):
                    d.start()

            @pl.loop(0, IC, step=NB)
            def _(j0):
                for b in range(NB):
                    j = j0 + b
                    gather(p, j, b).wait()
                    scatter(p, j, b).start(add=True)
                    jf = j + AH
                    bf = (b + AH) % NB

                    @pl.when(jf < IC)
                    def _():
                        @pl.when(jf >= NB)
                        def _():
                            scatter(p, 0, bf).wait()

                        gather(p, jf, bf).start()

            for b in range(NB):
                scatter(p, 0, b).wait()
            if c + 1 < NCH:
                for d in idx_fetch(c + 1, 1 - p):
                    d.wait()
                for b in range(AH):
                    gather(1 - p, b, b).start()

        plsc.subcore_barrier()
        pltpu.sync_copy(acc.at[pl.ds(r0, STRIPE)], out.at[cid].at[pl.ds(r0, STRIPE)])

    return edge_kernel


def _dinv_col(deg_ref, nrows):
    d = deg_ref[0, :nrows, 0:1] + deg_ref[1, :nrows, 0:1] + 1.0
    return lax.rsqrt(d)


def _tc_scale_matmul(x_ref, w_ref, deg_ref, out_ref):
    dinv = _dinv_col(deg_ref, N)
    h = jnp.dot(x_ref[...], w_ref[...], preferred_element_type=jnp.float32) * dinv
    h = h.astype(jnp.bfloat16)
    zpad = jnp.zeros((NP - N, HID_DIM // 2), jnp.bfloat16)
    out_ref[0] = jnp.concatenate([h[:, : HID_DIM // 2], zpad], axis=0)
    out_ref[1] = jnp.concatenate([h[:, HID_DIM // 2 :], zpad], axis=0)


def _tc_mid(t_ref, deg_ref, w_ref, b_ref, out_ref):
    dinv = _dinv_col(deg_ref, N)
    tmp = jnp.concatenate(
        [t_ref[0, :N].astype(jnp.float32), t_ref[1, :N].astype(jnp.float32)], axis=1
    )
    z = jnp.maximum(tmp * dinv + b_ref[...], 0.0)
    h = jnp.dot(z, w_ref[...], preferred_element_type=jnp.float32) * dinv
    h = h.astype(jnp.bfloat16)
    zpad = jnp.zeros((NP - N, OUT_DIM // 2), jnp.bfloat16)
    out_ref[0] = jnp.concatenate([h[:, : OUT_DIM // 2], zpad], axis=0)
    out_ref[1] = jnp.concatenate([h[:, OUT_DIM // 2 :], zpad], axis=0)


def _tc_final(t_ref, deg_ref, b_ref, out_ref):
    dinv = _dinv_col(deg_ref, N)
    o = jnp.concatenate(
        [t_ref[0, :N].astype(jnp.float32), t_ref[1, :N].astype(jnp.float32)], axis=1
    ) * dinv + b_ref[...]
    m = jnp.max(o, axis=1, keepdims=True)
    z = o - m
    out_ref[...] = z - jnp.log(jnp.sum(jnp.exp(z), axis=1, keepdims=True))


_scale_matmul_call = pl.pallas_call(
    _tc_scale_matmul,
    out_shape=jax.ShapeDtypeStruct((NC, NP, HID_DIM // 2), jnp.bfloat16),
)

_mid_call = pl.pallas_call(
    _tc_mid,
    out_shape=jax.ShapeDtypeStruct((NC, NP, OUT_DIM // 2), jnp.bfloat16),
)

_final_call = pl.pallas_call(
    _tc_final,
    out_shape=jax.ShapeDtypeStruct((N, OUT_DIM), jnp.float32),
)

def kernel(x, edge_index, W1, b1, W2, b2):
    ei = edge_index.astype(jnp.int32)
    pad = jnp.full((EPAD - E,), N, jnp.int32)
    src = jnp.concatenate([ei[0], pad]).reshape(ROWS, EROW)
    dst = jnp.concatenate([ei[1], pad]).reshape(ROWS, EROW)
    ones16 = jnp.ones((EROW, 16), jnp.float32)
    zeros16 = jnp.zeros((STRIPE, 16), jnp.float32)

    degp = _make_deg_kernel()(dst, ones16, zeros16)
    tbl1 = _scale_matmul_call(x, W1, degp)
    acc1 = _make_edge_kernel(HID_DIM // 2)(tbl1, src, dst)
    tbl2 = _mid_call(acc1, degp, W2, b1.reshape(1, HID_DIM))
    acc2 = _make_edge_kernel(OUT_DIM // 2)(tbl2, src, dst)
    return _final_call(acc2, degp, b2.reshape(1, OUT_DIM))
